# baseline (speedup 1.0000x reference)
import jax
import jax.numpy as jnp
from jax.experimental import pallas as pl

MAX_DEPTH = 20.0
VALID_THRESHOLD = 1e-06
MIN_VALID_POINTS = 10
BLOCK_B = 4


def _body(x_ref, y_ref, o_ref):
    h = x_ref.shape[1]
    x = x_ref[...]
    y = y_ref[...]
    mask = (y > VALID_THRESHOLD) & (y <= MAX_DEPTH)
    xm = jnp.where(mask, x, 0.0)
    ym = jnp.where(mask, y, 0.0)
    xxm = xm * xm
    xym = xm * ym
    ones = jnp.full((BLOCK_B, 8, h), 1.0, dtype=jnp.bfloat16)
    parts = [
        jax.lax.dot_general(
            ones, s.astype(jnp.bfloat16), (((2,), (1,)), ((0,), (0,))),
            preferred_element_type=jnp.float32,
        )
        for s in (xm, xxm, ym, xym)
    ]
    sums = jnp.sum(jnp.stack(parts), axis=(2, 3)) / 8.0
    n = jnp.sum(mask, axis=(1, 2)).astype(x.dtype)
    x_sum = sums[0]
    x_sq_sum = sums[1]
    y_sum = sums[2]
    xy_sum = sums[3]
    det = n * x_sq_sum - x_sum * x_sum
    valid = (n >= MIN_VALID_POINTS) & (jnp.abs(det) >= 1e-08)
    safe_det = jnp.where(valid, det, 1.0)
    scale = jnp.where(valid, (n * xy_sum - x_sum * y_sum) / safe_det, 1.0)
    bias = jnp.where(valid, (x_sq_sum * y_sum - x_sum * xy_sum) / safe_det, 0.0)
    o_ref[...] = scale[:, None, None] * x + bias[:, None, None]


def kernel(non_scale_dense, sparse_depth):
    b, c, h, w = non_scale_dense.shape
    x = non_scale_dense.reshape(b, h, w)
    y = sparse_depth.reshape(b, h, w)
    out = pl.pallas_call(
        _body,
        grid=(b // BLOCK_B,),
        in_specs=[
            pl.BlockSpec((BLOCK_B, h, w), lambda i: (i, 0, 0)),
            pl.BlockSpec((BLOCK_B, h, w), lambda i: (i, 0, 0)),
        ],
        out_specs=pl.BlockSpec((BLOCK_B, h, w), lambda i: (i, 0, 0)),
        out_shape=jax.ShapeDtypeStruct((b, h, w), x.dtype),
    )(x, y)
    return out.reshape(b, c, h, w)


# speedup vs baseline: 1.1322x; 1.1322x over previous
import jax
import jax.numpy as jnp
from jax.experimental import pallas as pl

MAX_DEPTH = 20.0
VALID_THRESHOLD = 1e-06
MIN_VALID_POINTS = 10
BLOCK_B = 4


def _body(x_ref, y_ref, o_ref):
    h = x_ref.shape[1]
    x = x_ref[...]
    y = y_ref[...]
    xb = x.astype(jnp.bfloat16)
    yb = y.astype(jnp.bfloat16)
    mask = (yb > VALID_THRESHOLD) & (yb <= MAX_DEPTH)
    zero = jnp.bfloat16(0.0)
    xm = jnp.where(mask, xb, zero)
    ym = jnp.where(mask, yb, zero)
    xxm = xm * xm
    xym = xm * ym
    mf = jnp.where(mask, jnp.bfloat16(1.0), zero)
    ones = jnp.full((BLOCK_B, 8, h), 1.0, dtype=jnp.bfloat16)
    parts = [
        jax.lax.dot_general(
            ones, s, (((2,), (1,)), ((0,), (0,))),
            preferred_element_type=jnp.float32,
        )
        for s in (mf, xm, xxm, ym, xym)
    ]
    sums = jnp.sum(jnp.stack(parts), axis=(2, 3)) / 8.0
    n = sums[0]
    x_sum = sums[1]
    x_sq_sum = sums[2]
    y_sum = sums[3]
    xy_sum = sums[4]
    det = n * x_sq_sum - x_sum * x_sum
    valid = (n >= MIN_VALID_POINTS) & (jnp.abs(det) >= 1e-08)
    safe_det = jnp.where(valid, det, 1.0)
    scale = jnp.where(valid, (n * xy_sum - x_sum * y_sum) / safe_det, 1.0)
    bias = jnp.where(valid, (x_sq_sum * y_sum - x_sum * xy_sum) / safe_det, 0.0)
    o_ref[...] = scale[:, None, None] * x + bias[:, None, None]


def kernel(non_scale_dense, sparse_depth):
    b, c, h, w = non_scale_dense.shape
    x = non_scale_dense.reshape(b, h, w)
    y = sparse_depth.reshape(b, h, w)
    out = pl.pallas_call(
        _body,
        grid=(b // BLOCK_B,),
        in_specs=[
            pl.BlockSpec((BLOCK_B, h, w), lambda i: (i, 0, 0)),
            pl.BlockSpec((BLOCK_B, h, w), lambda i: (i, 0, 0)),
        ],
        out_specs=pl.BlockSpec((BLOCK_B, h, w), lambda i: (i, 0, 0)),
        out_shape=jax.ShapeDtypeStruct((b, h, w), x.dtype),
    )(x, y)
    return out.reshape(b, c, h, w)
